# baseline (speedup 1.0000x reference)
import functools

import jax
import jax.numpy as jnp
from jax import lax
from jax.experimental import pallas as pl
from jax.experimental.pallas import tpu as pltpu
from jax.experimental.pallas import tpu_sc as plsc

S = 32768
E = 16
B = 32
STEP = 256
L = 16
CHUNK = 2048
UNROLL = 16
ACC = S + CHUNK - STEP


def _worker(x_hbm, idx_hbm, out_hbm, idx_v, acc, buf_a, buf_b, sem_a, sem_b):
    c = lax.axis_index("c")
    s = lax.axis_index("s")
    b = s * 2 + c

    pltpu.sync_copy(idx_hbm.at[b], idx_v)
    vec = idx_v[...]

    zeros = jnp.zeros((L,), jnp.float32)

    bufs = (buf_a, buf_b)
    sems = (sem_a, sem_b)
    starts = [vec[i] * STEP for i in range(E)]
    nblocks = [(S - starts[i] + CHUNK - 1) // CHUNK for i in range(E)]

    def stage(i, is_start):
        if i == 0:
            dst = lambda k: acc.at[pl.ds(starts[0] + k * CHUNK, CHUNK)]
        else:
            buf = bufs[i % 2]
            dst = lambda k: buf.at[pl.ds(k * CHUNK, CHUNK)]
        sem = sems[i % 2]

        def body(k, carry):
            copy = pltpu.make_async_copy(
                x_hbm.at[b, i, pl.ds(k * CHUNK, CHUNK)],
                dst(k),
                sem,
            )
            if is_start:
                copy.start()
            else:
                copy.wait()
            return carry

        lax.fori_loop(0, nblocks[i], body, None)

    stage(0, True)
    st0 = starts[0]

    def zero_block(k, carry):
        base = k * STEP

        @plsc.parallel_loop(0, STEP, step=L, unroll=UNROLL)
        def _zero(j):
            acc[pl.ds(base + j, L)] = zeros

        return carry

    lax.fori_loop(0, st0 // STEP, zero_block, None)

    for i in range(E):
        if i + 1 < E:
            stage(i + 1, True)
        if i == 0:
            stage(i, False)
            continue
        st = starts[i]
        cur = bufs[i % 2]
        sem = sems[i % 2]

        def wait_add_block(k, carry):
            base = k * CHUNK
            pltpu.make_async_copy(
                x_hbm.at[b, i, pl.ds(base, CHUNK)],
                cur.at[pl.ds(base, CHUNK)],
                sem,
            ).wait()

            @plsc.parallel_loop(0, CHUNK, step=L, unroll=UNROLL)
            def _add(j):
                plsc.addupdate(acc.at[pl.ds(st + base + j, L)],
                               cur[pl.ds(base + j, L)])

            return carry

        lax.fori_loop(0, nblocks[i], wait_add_block, None)

    pltpu.sync_copy(acc.at[pl.ds(0, S)], out_hbm.at[b, 0])


_mesh = plsc.VectorSubcoreMesh(core_axis_name="c", subcore_axis_name="s")

_summarize = functools.partial(
    pl.kernel,
    mesh=_mesh,
    out_type=jax.ShapeDtypeStruct((B, 1, S), jnp.float32),
    scratch_types=[
        pltpu.VMEM((E,), jnp.int32),
        pltpu.VMEM((ACC,), jnp.float32),
        pltpu.VMEM((S,), jnp.float32),
        pltpu.VMEM((S,), jnp.float32),
        pltpu.SemaphoreType.DMA,
        pltpu.SemaphoreType.DMA,
    ],
)(_worker)


def kernel(x, indices):
    return _summarize(x, indices.astype(jnp.int32))


# speedup vs baseline: 1.0246x; 1.0246x over previous
import functools

import jax
import jax.numpy as jnp
from jax import lax
from jax.experimental import pallas as pl
from jax.experimental.pallas import tpu as pltpu
from jax.experimental.pallas import tpu_sc as plsc

S = 32768
E = 16
B = 32
STEP = 256
L = 16
CHUNK = 2048
UNROLL = 8
ACC = S + CHUNK - STEP


def _worker(x_hbm, idx_hbm, out_hbm, idx_v, acc, buf_a, buf_b, sem_a, sem_b):
    c = lax.axis_index("c")
    s = lax.axis_index("s")
    b = s * 2 + c

    pltpu.sync_copy(idx_hbm.at[b], idx_v)
    vec = idx_v[...]

    zeros = jnp.zeros((L,), jnp.float32)

    bufs = (buf_a, buf_b)
    sems = (sem_a, sem_b)
    starts = [vec[i] * STEP for i in range(E)]
    nblocks = [(S - starts[i] + CHUNK - 1) // CHUNK for i in range(E)]

    def stage(i, is_start):
        if i == 0:
            dst = lambda k: acc.at[pl.ds(starts[0] + k * CHUNK, CHUNK)]
        else:
            buf = bufs[i % 2]
            dst = lambda k: buf.at[pl.ds(k * CHUNK, CHUNK)]
        sem = sems[i % 2]

        def body(k, carry):
            copy = pltpu.make_async_copy(
                x_hbm.at[b, i, pl.ds(k * CHUNK, CHUNK)],
                dst(k),
                sem,
            )
            if is_start:
                copy.start()
            else:
                copy.wait()
            return carry

        lax.fori_loop(0, nblocks[i], body, None)

    stage(0, True)
    st0 = starts[0]

    def zero_block(k, carry):
        base = k * STEP

        @plsc.parallel_loop(0, STEP, step=L, unroll=UNROLL)
        def _zero(j):
            acc[pl.ds(base + j, L)] = zeros

        return carry

    lax.fori_loop(0, st0 // STEP, zero_block, None)

    for i in range(E):
        if i + 1 < E:
            stage(i + 1, True)
        if i == 0:
            stage(i, False)
            continue
        st = starts[i]
        cur = bufs[i % 2]
        sem = sems[i % 2]

        def wait_add_block(k, carry):
            base = k * CHUNK
            pltpu.make_async_copy(
                x_hbm.at[b, i, pl.ds(base, CHUNK)],
                cur.at[pl.ds(base, CHUNK)],
                sem,
            ).wait()

            @plsc.parallel_loop(0, CHUNK, step=L, unroll=UNROLL)
            def _add(j):
                plsc.addupdate(acc.at[pl.ds(st + base + j, L)],
                               cur[pl.ds(base + j, L)])

            return carry

        lax.fori_loop(0, nblocks[i], wait_add_block, None)

    pltpu.sync_copy(acc.at[pl.ds(0, S)], out_hbm.at[b, 0])


_mesh = plsc.VectorSubcoreMesh(core_axis_name="c", subcore_axis_name="s")

_summarize = functools.partial(
    pl.kernel,
    mesh=_mesh,
    out_type=jax.ShapeDtypeStruct((B, 1, S), jnp.float32),
    scratch_types=[
        pltpu.VMEM((E,), jnp.int32),
        pltpu.VMEM((ACC,), jnp.float32),
        pltpu.VMEM((S,), jnp.float32),
        pltpu.VMEM((S,), jnp.float32),
        pltpu.SemaphoreType.DMA,
        pltpu.SemaphoreType.DMA,
    ],
)(_worker)


def kernel(x, indices):
    return _summarize(x, indices.astype(jnp.int32))
